# baseline (speedup 1.0000x reference)
import functools

import jax
import jax.numpy as jnp
from jax import lax
from jax.experimental import pallas as pl
from jax.experimental.pallas import tpu as pltpu
from jax.experimental.pallas import tpu_sc as plsc

N = 10000
E = 320000
D = 128
NCLS = 40

_TILES = 32
_NS = 16
_EPT = E // _TILES
_CH = 80
_NCHUNK = _EPT // _CH

_ACH = 80
_W = 8
_PAD = 240
_EPTP = _EPT + _PAD
_ANCH = _EPTP // _ACH
_NG = _ANCH // _W
_NB = 4
_NP = N + _NS

_HI = jax.lax.Precision.HIGHEST


@functools.cache
def _sc_agg():
    mesh = plsc.VectorSubcoreMesh(core_axis_name="c", subcore_axis_name="s")

    @functools.partial(
        pl.kernel,
        out_type=jax.ShapeDtypeStruct((2, _NP, D), jnp.float32),
        mesh=mesh,
        scratch_types=[
            pltpu.VMEM((_W, _ACH), jnp.int32),
            pltpu.VMEM((_W, _ACH), jnp.int32),
            pltpu.VMEM((_W, _ACH), jnp.int32),
            pltpu.VMEM((_W, _ACH), jnp.int32),
            pltpu.VMEM((_ACH, D), jnp.float32),
            pltpu.VMEM((_ACH, D), jnp.float32),
            pltpu.VMEM((_ACH, D), jnp.float32),
            pltpu.VMEM((_ACH, D), jnp.float32),
            pltpu.VMEM_SHARED((_NP, D), jnp.float32),
            pltpu.SemaphoreType.DMA((_NB,)),
            pltpu.SemaphoreType.DMA((_NB,)),
            pltpu.SemaphoreType.DMA((2,)),
        ],
    )
    def agg(p_hbm, srcr_hbm, dstr_hbm, zmat_hbm, out_hbm,
            si0, si1, di0, di1, rb0, rb1, rb2, rb3, acc, gsem, ssem, isem):
        sibs = [si0, si1]
        dibs = [di0, di1]
        rbs = [rb0, rb1, rb2, rb3]
        c = lax.axis_index("c")
        s = lax.axis_index("s")
        t = c * _NS + s

        @pl.when(s == 0)
        def _():
            pltpu.sync_copy(zmat_hbm, acc)

        plsc.subcore_barrier()

        def g_start(k):
            g, j, b = k // _W, k % _W, k % _NB
            pltpu.async_copy(p_hbm.at[sibs[g % 2].at[j]], rbs[b], gsem.at[b])

        def g_wait(k):
            g, j, b = k // _W, k % _W, k % _NB
            pltpu.make_async_copy(p_hbm.at[sibs[g % 2].at[j]], rbs[b],
                                  gsem.at[b]).wait()

        def s_start(k):
            g, j, b = k // _W, k % _W, k % _NB
            pltpu.async_copy(rbs[b], acc.at[dibs[g % 2].at[j]], ssem.at[b],
                             add=True)

        def s_wait(k):
            g, j, b = k // _W, k % _W, k % _NB
            pltpu.make_async_copy(rbs[b], acc.at[dibs[g % 2].at[j]],
                                  ssem.at[b]).wait()

        def i_start(g):
            ib = g % 2
            pltpu.async_copy(srcr_hbm.at[t, g], sibs[ib], isem.at[ib])
            pltpu.async_copy(dstr_hbm.at[t, g], dibs[ib], isem.at[ib])

        def i_wait(g):
            ib = g % 2
            pltpu.make_async_copy(srcr_hbm.at[t, g], sibs[ib],
                                  isem.at[ib]).wait()
            pltpu.make_async_copy(dstr_hbm.at[t, g], dibs[ib],
                                  isem.at[ib]).wait()

        pltpu.sync_copy(srcr_hbm.at[t, 0], si0)
        pltpu.sync_copy(dstr_hbm.at[t, 0], di0)
        i_start(1)
        for k in range(_NB - 1):
            g_start(k)

        for k in range(_ANCH):
            g_wait(k)
            s_start(k)
            if k > 0:
                s_wait(k - 1)
            if k % _W == 0 and k > 0 and k // _W + 1 < _NG:
                i_start(k // _W + 1)
            kn = k + _NB - 1
            if kn < _ANCH:
                if kn % _W < _NB - 1 and kn // _W > 0:
                    if kn % _W == 0:
                        i_wait(kn // _W)
                g_start(kn)

        s_wait(_ANCH - 1)
        plsc.subcore_barrier()

        @pl.when(s == 0)
        def _():
            pltpu.sync_copy(acc, out_hbm.at[c])

    return agg


@functools.cache
def _sc_deg():
    mesh = plsc.VectorSubcoreMesh(core_axis_name="c", subcore_axis_name="s")

    @functools.partial(
        pl.kernel,
        out_type=jax.ShapeDtypeStruct((2, _NP), jnp.float32),
        mesh=mesh,
        scratch_types=[
            pltpu.VMEM((_W, _ACH), jnp.int32),
            pltpu.VMEM((_W, _ACH), jnp.int32),
            pltpu.VMEM((_ACH,), jnp.float32),
            pltpu.VMEM_SHARED((_NP,), jnp.float32),
            pltpu.SemaphoreType.DMA((2,)),
        ],
    )
    def deg(dstr_hbm, zvec_hbm, out_hbm, di0, di1, ones_v, acc, isem):
        dibs = [di0, di1]
        c = lax.axis_index("c")
        s = lax.axis_index("s")
        t = c * _NS + s

        @pl.when(s == 0)
        def _():
            pltpu.sync_copy(zvec_hbm, acc)

        for i in range(_ACH // 16):
            ones_v[pl.ds(i * 16, 16)] = jnp.full((16,), 1.0, jnp.float32)

        plsc.subcore_barrier()
        pltpu.sync_copy(dstr_hbm.at[t, 0], di0)

        def i_start(g):
            pltpu.async_copy(dstr_hbm.at[t, g], dibs[g % 2], isem.at[g % 2])

        def i_wait(g):
            pltpu.make_async_copy(dstr_hbm.at[t, g], dibs[g % 2],
                                  isem.at[g % 2]).wait()

        i_start(1)
        for g in range(_NG):
            if g > 0:
                i_wait(g)
            for j in range(_W):
                pltpu.sync_copy(ones_v, acc.at[dibs[g % 2].at[j]], add=True)
            if g + 2 < _NG:
                i_start(g + 2)

        plsc.subcore_barrier()

        @pl.when(s == 0)
        def _():
            pltpu.sync_copy(acc, out_hbm.at[c])

    return deg


def _tc1_body(x_ref, w_ref, dega_ref, degb_ref, p_ref, disnl_ref, diswl_ref):
    deg = dega_ref[...] + degb_ref[...]
    disnl = jnp.where(deg > 0, lax.rsqrt(jnp.maximum(deg, 1e-12)), 0.0)
    diswl = lax.rsqrt(deg + 1.0)
    disnl_ref[...] = disnl
    diswl_ref[...] = diswl
    p_ref[...] = disnl * jnp.dot(
        x_ref[...], w_ref[...], preferred_element_type=jnp.float32, precision=_HI
    )


def _tc2_body(a0_ref, a1_ref, disnl_ref, b_ref, w_ref, p_ref):
    disnl = disnl_ref[...]
    x1 = jnp.maximum(disnl * (a0_ref[...] + a1_ref[...]) + b_ref[...], 0.0)
    p_ref[...] = disnl * jnp.dot(
        x1, w_ref[...], preferred_element_type=jnp.float32, precision=_HI
    )


def _tc3_body(a0_ref, a1_ref, disnl_ref, diswl_ref, b_ref, p_ref):
    x2 = jnp.maximum(
        disnl_ref[...] * (a0_ref[...] + a1_ref[...]) + b_ref[...], 0.0
    )
    p_ref[...] = diswl_ref[...] * x2


def _tc4_body(a0_ref, a1_ref, p2_ref, diswl_ref, w_ref, b_ref, o_ref):
    q = diswl_ref[...] * (a0_ref[...] + a1_ref[...] + p2_ref[...])
    o_ref[...] = (
        jnp.dot(q, w_ref[...], preferred_element_type=jnp.float32, precision=_HI)
        + b_ref[...]
    )


def _call(body, n_out, out_shapes):
    return pl.pallas_call(
        body,
        out_shape=[jax.ShapeDtypeStruct(s, jnp.float32) for s in out_shapes]
        if n_out > 1
        else jax.ShapeDtypeStruct(out_shapes[0], jnp.float32),
    )


def kernel(x, edge_index, W0, b0, W1, b1, W2, b2):
    dsrc = (jnp.arange(_TILES, dtype=jnp.int32) * 313 % N)[:, None]
    ddst = (N + jnp.arange(_TILES, dtype=jnp.int32) % _NS)[:, None]
    srcr = jnp.concatenate(
        [edge_index[0].reshape(_TILES, _EPT),
         jnp.broadcast_to(dsrc, (_TILES, _PAD))], axis=1
    ).reshape(_TILES, _NG, _W, _ACH)
    dstr = jnp.concatenate(
        [edge_index[1].reshape(_TILES, _EPT),
         jnp.broadcast_to(ddst, (_TILES, _PAD))], axis=1
    ).reshape(_TILES, _NG, _W, _ACH)
    zvec = jnp.zeros((_NP,), jnp.float32)
    zmat = jnp.zeros((_NP, D), jnp.float32)

    degp = _sc_deg()(dstr, zvec)
    dega = degp[0][:N, None]
    degb = degp[1][:N, None]

    p0, disnl, diswl = _call(_tc1_body, 3, [(N, D), (N, 1), (N, 1)])(
        x, W0, dega, degb
    )

    a1 = _sc_agg()(p0, srcr, dstr, zmat)
    p1 = _call(_tc2_body, 1, [(N, D)])(
        a1[0, :N], a1[1, :N], disnl, b0[None, :], W1
    )

    a2 = _sc_agg()(p1, srcr, dstr, zmat)
    p2 = _call(_tc3_body, 1, [(N, D)])(
        a2[0, :N], a2[1, :N], disnl, diswl, b1[None, :]
    )

    a3 = _sc_agg()(p2, srcr, dstr, zmat)
    out = _call(_tc4_body, 1, [(N, NCLS)])(
        a3[0, :N], a3[1, :N], p2, diswl, W2, b2[None, :]
    )
    return out


# speedup vs baseline: 1.2150x; 1.2150x over previous
import functools

import jax
import jax.numpy as jnp
from jax import lax
from jax.experimental import pallas as pl
from jax.experimental.pallas import tpu as pltpu
from jax.experimental.pallas import tpu_sc as plsc

N = 10000
E = 320000
D = 128
NCLS = 40

_TILES = 32
_NS = 16
_EPT = E // _TILES
_CH = 80
_NCHUNK = _EPT // _CH

_ACH = 80
_W = 8
_PAD = 240
_EPTP = _EPT + _PAD
_ANCH = _EPTP // _ACH
_NG = _ANCH // _W
_NB = 4
_NP = N + _NS

_HI = jax.lax.Precision.HIGHEST


@functools.cache
def _sc_agg():
    mesh = plsc.VectorSubcoreMesh(core_axis_name="c", subcore_axis_name="s")

    @functools.partial(
        pl.kernel,
        out_type=jax.ShapeDtypeStruct((2, _NP, D), jnp.float32),
        mesh=mesh,
        scratch_types=[
            pltpu.VMEM((_W, _ACH), jnp.int32),
            pltpu.VMEM((_W, _ACH), jnp.int32),
            pltpu.VMEM((_W, _ACH), jnp.int32),
            pltpu.VMEM((_W, _ACH), jnp.int32),
            pltpu.VMEM((_ACH, D), jnp.float32),
            pltpu.VMEM((_ACH, D), jnp.float32),
            pltpu.VMEM((_ACH, D), jnp.float32),
            pltpu.VMEM((_ACH, D), jnp.float32),
            pltpu.VMEM_SHARED((_NP, D), jnp.float32),
            pltpu.SemaphoreType.DMA((_NB,)),
            pltpu.SemaphoreType.DMA((_NB,)),
            pltpu.SemaphoreType.DMA((2,)),
        ],
    )
    def agg(p_hbm, srcr_hbm, dstr_hbm, zmat_hbm, out_hbm,
            si0, si1, di0, di1, rb0, rb1, rb2, rb3, acc, gsem, ssem, isem):
        sibs = [si0, si1]
        dibs = [di0, di1]
        rbs = [rb0, rb1, rb2, rb3]
        c = lax.axis_index("c")
        s = lax.axis_index("s")
        t = c * _NS + s

        for i in range(_NS):
            @pl.when(s == i)
            def _():
                lo = 624 * i
                sz = _NP - lo if i == _NS - 1 else 624
                pltpu.sync_copy(zmat_hbm.at[pl.ds(lo, sz)],
                                acc.at[pl.ds(lo, sz)])

        plsc.subcore_barrier()

        def g_start(k):
            g, j, b = k // _W, k % _W, k % _NB
            pltpu.async_copy(p_hbm.at[sibs[g % 2].at[j]], rbs[b], gsem.at[b])

        def g_wait(k):
            g, j, b = k // _W, k % _W, k % _NB
            pltpu.make_async_copy(p_hbm.at[sibs[g % 2].at[j]], rbs[b],
                                  gsem.at[b]).wait()

        def s_start(k):
            g, j, b = k // _W, k % _W, k % _NB
            pltpu.async_copy(rbs[b], acc.at[dibs[g % 2].at[j]], ssem.at[b],
                             add=True)

        def s_wait(k):
            g, j, b = k // _W, k % _W, k % _NB
            pltpu.make_async_copy(rbs[b], acc.at[dibs[g % 2].at[j]],
                                  ssem.at[b]).wait()

        def i_start(g):
            ib = g % 2
            pltpu.async_copy(srcr_hbm.at[t, g], sibs[ib], isem.at[ib])
            pltpu.async_copy(dstr_hbm.at[t, g], dibs[ib], isem.at[ib])

        def i_wait(g):
            ib = g % 2
            pltpu.make_async_copy(srcr_hbm.at[t, g], sibs[ib],
                                  isem.at[ib]).wait()
            pltpu.make_async_copy(dstr_hbm.at[t, g], dibs[ib],
                                  isem.at[ib]).wait()

        pltpu.sync_copy(srcr_hbm.at[t, 0], si0)
        pltpu.sync_copy(dstr_hbm.at[t, 0], di0)
        i_start(1)
        for k in range(_NB - 1):
            g_start(k)

        for k in range(_ANCH):
            g_wait(k)
            s_start(k)
            if k > 0:
                s_wait(k - 1)
            if k % _W == 0 and k > 0 and k // _W + 1 < _NG:
                i_start(k // _W + 1)
            kn = k + _NB - 1
            if kn < _ANCH:
                if kn % _W < _NB - 1 and kn // _W > 0:
                    if kn % _W == 0:
                        i_wait(kn // _W)
                g_start(kn)

        s_wait(_ANCH - 1)
        plsc.subcore_barrier()

        for i in range(_NS):
            @pl.when(s == i)
            def _():
                lo = 624 * i
                sz = _NP - lo if i == _NS - 1 else 624
                pltpu.sync_copy(acc.at[pl.ds(lo, sz)],
                                out_hbm.at[c, pl.ds(lo, sz)])

    return agg


@functools.cache
def _sc_deg():
    mesh = plsc.VectorSubcoreMesh(core_axis_name="c", subcore_axis_name="s")

    @functools.partial(
        pl.kernel,
        out_type=jax.ShapeDtypeStruct((2, _NP), jnp.float32),
        mesh=mesh,
        scratch_types=[
            pltpu.VMEM((_W, _ACH), jnp.int32),
            pltpu.VMEM((_W, _ACH), jnp.int32),
            pltpu.VMEM((_ACH,), jnp.float32),
            pltpu.VMEM_SHARED((_NP,), jnp.float32),
            pltpu.SemaphoreType.DMA((2,)),
        ],
    )
    def deg(dstr_hbm, zvec_hbm, out_hbm, di0, di1, ones_v, acc, isem):
        dibs = [di0, di1]
        c = lax.axis_index("c")
        s = lax.axis_index("s")
        t = c * _NS + s

        @pl.when(s == 0)
        def _():
            pltpu.sync_copy(zvec_hbm, acc)

        for i in range(_ACH // 16):
            ones_v[pl.ds(i * 16, 16)] = jnp.full((16,), 1.0, jnp.float32)

        plsc.subcore_barrier()
        pltpu.sync_copy(dstr_hbm.at[t, 0], di0)

        def i_start(g):
            pltpu.async_copy(dstr_hbm.at[t, g], dibs[g % 2], isem.at[g % 2])

        def i_wait(g):
            pltpu.make_async_copy(dstr_hbm.at[t, g], dibs[g % 2],
                                  isem.at[g % 2]).wait()

        i_start(1)
        for g in range(_NG):
            if g > 0:
                i_wait(g)
            for j in range(_W):
                pltpu.sync_copy(ones_v, acc.at[dibs[g % 2].at[j]], add=True)
            if g + 2 < _NG:
                i_start(g + 2)

        plsc.subcore_barrier()

        @pl.when(s == 0)
        def _():
            pltpu.sync_copy(acc, out_hbm.at[c])

    return deg


def _tc1_body(x_ref, w_ref, degp_ref, p_ref, disnl_ref, diswl_ref):
    deg = degp_ref[0, :N, :] + degp_ref[1, :N, :]
    disnl = jnp.where(deg > 0, lax.rsqrt(jnp.maximum(deg, 1e-12)), 0.0)
    diswl = lax.rsqrt(deg + 1.0)
    disnl_ref[...] = disnl
    diswl_ref[...] = diswl
    p_ref[...] = disnl * jnp.dot(
        x_ref[...], w_ref[...], preferred_element_type=jnp.float32, precision=_HI
    )


def _tc2_body(a_ref, disnl_ref, b_ref, w_ref, p_ref):
    disnl = disnl_ref[...]
    x1 = jnp.maximum(
        disnl * (a_ref[0, :N, :] + a_ref[1, :N, :]) + b_ref[...], 0.0
    )
    p_ref[...] = disnl * jnp.dot(
        x1, w_ref[...], preferred_element_type=jnp.float32, precision=_HI
    )


def _tc3_body(a_ref, disnl_ref, diswl_ref, b_ref, p_ref):
    x2 = jnp.maximum(
        disnl_ref[...] * (a_ref[0, :N, :] + a_ref[1, :N, :]) + b_ref[...], 0.0
    )
    p_ref[...] = diswl_ref[...] * x2


def _tc4_body(a_ref, p2_ref, diswl_ref, w_ref, b_ref, o_ref):
    q = diswl_ref[...] * (a_ref[0, :N, :] + a_ref[1, :N, :] + p2_ref[...])
    o_ref[...] = (
        jnp.dot(q, w_ref[...], preferred_element_type=jnp.float32, precision=_HI)
        + b_ref[...]
    )


def _call(body, n_out, out_shapes):
    return pl.pallas_call(
        body,
        out_shape=[jax.ShapeDtypeStruct(s, jnp.float32) for s in out_shapes]
        if n_out > 1
        else jax.ShapeDtypeStruct(out_shapes[0], jnp.float32),
    )


def kernel(x, edge_index, W0, b0, W1, b1, W2, b2):
    tvec = jnp.arange(_TILES, dtype=jnp.int32)[:, None]
    evec = jnp.arange(_PAD, dtype=jnp.int32)[None, :]
    dsrc = (tvec * 313 + evec * 41) % N
    ddst = N + (tvec + evec) % _NS
    srcr = jnp.concatenate(
        [edge_index[0].reshape(_TILES, _EPT), dsrc], axis=1
    ).reshape(_TILES, _NG, _W, _ACH)
    dstr = jnp.concatenate(
        [edge_index[1].reshape(_TILES, _EPT), ddst], axis=1
    ).reshape(_TILES, _NG, _W, _ACH)
    zvec = jnp.zeros((_NP,), jnp.float32)
    zmat = jnp.zeros((_NP, D), jnp.float32)

    degp = _sc_deg()(dstr, zvec)

    p0, disnl, diswl = _call(_tc1_body, 3, [(N, D), (N, 1), (N, 1)])(
        x, W0, degp.reshape(2, _NP, 1)
    )

    a1 = _sc_agg()(p0, srcr, dstr, zmat)
    p1 = _call(_tc2_body, 1, [(N, D)])(a1, disnl, b0[None, :], W1)

    a2 = _sc_agg()(p1, srcr, dstr, zmat)
    p2 = _call(_tc3_body, 1, [(N, D)])(a2, disnl, diswl, b1[None, :])

    a3 = _sc_agg()(p2, srcr, dstr, zmat)
    out = _call(_tc4_body, 1, [(N, NCLS)])(a3, p2, diswl, W2, b2[None, :])
    return out


# speedup vs baseline: 1.2477x; 1.0269x over previous
import functools

import jax
import jax.numpy as jnp
from jax import lax
from jax.experimental import pallas as pl
from jax.experimental.pallas import tpu as pltpu
from jax.experimental.pallas import tpu_sc as plsc

N = 10000
E = 320000
D = 128
NCLS = 40

_TILES = 32
_NS = 16
_EPT = E // _TILES
_CH = 80
_NCHUNK = _EPT // _CH

_ACH = 64
_W = 8
_PAD = 240
_EPTP = _EPT + _PAD
_ANCH = _EPTP // _ACH
_NG = _ANCH // _W
_NB = 5
_NP = N + _NS

_HI = jax.lax.Precision.HIGHEST


@functools.cache
def _sc_agg():
    mesh = plsc.VectorSubcoreMesh(core_axis_name="c", subcore_axis_name="s")

    @functools.partial(
        pl.kernel,
        out_type=jax.ShapeDtypeStruct((2, _NP, D), jnp.float32),
        mesh=mesh,
        scratch_types=[
            pltpu.VMEM((_W, _ACH), jnp.int32),
            pltpu.VMEM((_W, _ACH), jnp.int32),
            pltpu.VMEM((_W, _ACH), jnp.int32),
            pltpu.VMEM((_W, _ACH), jnp.int32),
            pltpu.VMEM((_ACH, D), jnp.float32),
            pltpu.VMEM((_ACH, D), jnp.float32),
            pltpu.VMEM((_ACH, D), jnp.float32),
            pltpu.VMEM((_ACH, D), jnp.float32),
            pltpu.VMEM((_ACH, D), jnp.float32),
            pltpu.VMEM_SHARED((_NP, D), jnp.float32),
            pltpu.SemaphoreType.DMA((_NB,)),
            pltpu.SemaphoreType.DMA((_NB,)),
            pltpu.SemaphoreType.DMA((2,)),
        ],
    )
    def agg(p_hbm, srcr_hbm, dstr_hbm, zmat_hbm, out_hbm,
            si0, si1, di0, di1, rb0, rb1, rb2, rb3, rb4, acc, gsem, ssem, isem):
        sibs = [si0, si1]
        dibs = [di0, di1]
        rbs = [rb0, rb1, rb2, rb3, rb4]
        c = lax.axis_index("c")
        s = lax.axis_index("s")
        t = c * _NS + s

        for i in range(_NS):
            @pl.when(s == i)
            def _():
                lo = 624 * i
                sz = _NP - lo if i == _NS - 1 else 624
                pltpu.sync_copy(zmat_hbm.at[pl.ds(lo, sz)],
                                acc.at[pl.ds(lo, sz)])

        plsc.subcore_barrier()

        def g_start(k):
            g, j, b = k // _W, k % _W, k % _NB
            pltpu.async_copy(p_hbm.at[sibs[g % 2].at[j]], rbs[b], gsem.at[b])

        def g_wait(k):
            g, j, b = k // _W, k % _W, k % _NB
            pltpu.make_async_copy(p_hbm.at[sibs[g % 2].at[j]], rbs[b],
                                  gsem.at[b]).wait()

        def s_start(k):
            g, j, b = k // _W, k % _W, k % _NB
            pltpu.async_copy(rbs[b], acc.at[dibs[g % 2].at[j]], ssem.at[b],
                             add=True)

        def s_wait(k):
            g, j, b = k // _W, k % _W, k % _NB
            pltpu.make_async_copy(rbs[b], acc.at[dibs[g % 2].at[j]],
                                  ssem.at[b]).wait()

        def i_start(g):
            ib = g % 2
            pltpu.async_copy(srcr_hbm.at[t, g], sibs[ib], isem.at[ib])
            pltpu.async_copy(dstr_hbm.at[t, g], dibs[ib], isem.at[ib])

        def i_wait(g):
            ib = g % 2
            pltpu.make_async_copy(srcr_hbm.at[t, g], sibs[ib],
                                  isem.at[ib]).wait()
            pltpu.make_async_copy(dstr_hbm.at[t, g], dibs[ib],
                                  isem.at[ib]).wait()

        pltpu.sync_copy(srcr_hbm.at[t, 0], si0)
        pltpu.sync_copy(dstr_hbm.at[t, 0], di0)
        i_start(1)
        for k in range(_NB - 1):
            g_start(k)

        for k in range(_ANCH):
            g_wait(k)
            s_start(k)
            if k > 0:
                s_wait(k - 1)
            if k % _W == 0 and k > 0 and k // _W + 1 < _NG:
                i_start(k // _W + 1)
            kn = k + _NB - 1
            if kn < _ANCH:
                if kn % _W < _NB - 1 and kn // _W > 0:
                    if kn % _W == 0:
                        i_wait(kn // _W)
                g_start(kn)

        s_wait(_ANCH - 1)
        plsc.subcore_barrier()

        for i in range(_NS):
            @pl.when(s == i)
            def _():
                lo = 624 * i
                sz = _NP - lo if i == _NS - 1 else 624
                pltpu.sync_copy(acc.at[pl.ds(lo, sz)],
                                out_hbm.at[c, pl.ds(lo, sz)])

    return agg


@functools.cache
def _sc_deg():
    mesh = plsc.VectorSubcoreMesh(core_axis_name="c", subcore_axis_name="s")

    @functools.partial(
        pl.kernel,
        out_type=jax.ShapeDtypeStruct((2, _NP), jnp.float32),
        mesh=mesh,
        scratch_types=[
            pltpu.VMEM((_W, _ACH), jnp.int32),
            pltpu.VMEM((_W, _ACH), jnp.int32),
            pltpu.VMEM((_ACH,), jnp.float32),
            pltpu.VMEM_SHARED((_NP,), jnp.float32),
            pltpu.SemaphoreType.DMA((2,)),
        ],
    )
    def deg(dstr_hbm, zvec_hbm, out_hbm, di0, di1, ones_v, acc, isem):
        dibs = [di0, di1]
        c = lax.axis_index("c")
        s = lax.axis_index("s")
        t = c * _NS + s

        @pl.when(s == 0)
        def _():
            pltpu.sync_copy(zvec_hbm, acc)

        for i in range(_ACH // 16):
            ones_v[pl.ds(i * 16, 16)] = jnp.full((16,), 1.0, jnp.float32)

        plsc.subcore_barrier()
        pltpu.sync_copy(dstr_hbm.at[t, 0], di0)

        def i_start(g):
            pltpu.async_copy(dstr_hbm.at[t, g], dibs[g % 2], isem.at[g % 2])

        def i_wait(g):
            pltpu.make_async_copy(dstr_hbm.at[t, g], dibs[g % 2],
                                  isem.at[g % 2]).wait()

        i_start(1)
        for g in range(_NG):
            if g > 0:
                i_wait(g)
            for j in range(_W):
                pltpu.sync_copy(ones_v, acc.at[dibs[g % 2].at[j]], add=True)
            if g + 2 < _NG:
                i_start(g + 2)

        plsc.subcore_barrier()

        @pl.when(s == 0)
        def _():
            pltpu.sync_copy(acc, out_hbm.at[c])

    return deg


def _tc1_body(x_ref, w_ref, degp_ref, p_ref, disnl_ref, diswl_ref):
    deg = degp_ref[0, :N, :] + degp_ref[1, :N, :]
    disnl = jnp.where(deg > 0, lax.rsqrt(jnp.maximum(deg, 1e-12)), 0.0)
    diswl = lax.rsqrt(deg + 1.0)
    disnl_ref[...] = disnl
    diswl_ref[...] = diswl
    p_ref[...] = disnl * jnp.dot(
        x_ref[...], w_ref[...], preferred_element_type=jnp.float32, precision=_HI
    )


def _tc2_body(a_ref, disnl_ref, b_ref, w_ref, p_ref):
    disnl = disnl_ref[...]
    x1 = jnp.maximum(
        disnl * (a_ref[0, :N, :] + a_ref[1, :N, :]) + b_ref[...], 0.0
    )
    p_ref[...] = disnl * jnp.dot(
        x1, w_ref[...], preferred_element_type=jnp.float32, precision=_HI
    )


def _tc3_body(a_ref, disnl_ref, diswl_ref, b_ref, p_ref):
    x2 = jnp.maximum(
        disnl_ref[...] * (a_ref[0, :N, :] + a_ref[1, :N, :]) + b_ref[...], 0.0
    )
    p_ref[...] = diswl_ref[...] * x2


def _tc4_body(a_ref, p2_ref, diswl_ref, w_ref, b_ref, o_ref):
    q = diswl_ref[...] * (a_ref[0, :N, :] + a_ref[1, :N, :] + p2_ref[...])
    o_ref[...] = (
        jnp.dot(q, w_ref[...], preferred_element_type=jnp.float32, precision=_HI)
        + b_ref[...]
    )


def _call(body, n_out, out_shapes):
    return pl.pallas_call(
        body,
        out_shape=[jax.ShapeDtypeStruct(s, jnp.float32) for s in out_shapes]
        if n_out > 1
        else jax.ShapeDtypeStruct(out_shapes[0], jnp.float32),
    )


def kernel(x, edge_index, W0, b0, W1, b1, W2, b2):
    tvec = jnp.arange(_TILES, dtype=jnp.int32)[:, None]
    evec = jnp.arange(_PAD, dtype=jnp.int32)[None, :]
    dsrc = (tvec * 313 + evec * 41) % N
    ddst = N + (tvec + evec) % _NS
    srcr = jnp.concatenate(
        [edge_index[0].reshape(_TILES, _EPT), dsrc], axis=1
    ).reshape(_TILES, _NG, _W, _ACH)
    dstr = jnp.concatenate(
        [edge_index[1].reshape(_TILES, _EPT), ddst], axis=1
    ).reshape(_TILES, _NG, _W, _ACH)
    zvec = jnp.zeros((_NP,), jnp.float32)
    zmat = jnp.zeros((_NP, D), jnp.float32)

    degp = _sc_deg()(dstr, zvec)

    p0, disnl, diswl = _call(_tc1_body, 3, [(N, D), (N, 1), (N, 1)])(
        x, W0, degp.reshape(2, _NP, 1)
    )

    a1 = _sc_agg()(p0, srcr, dstr, zmat)
    p1 = _call(_tc2_body, 1, [(N, D)])(a1, disnl, b0[None, :], W1)

    a2 = _sc_agg()(p1, srcr, dstr, zmat)
    p2 = _call(_tc3_body, 1, [(N, D)])(a2, disnl, diswl, b1[None, :])

    a3 = _sc_agg()(p2, srcr, dstr, zmat)
    out = _call(_tc4_body, 1, [(N, NCLS)])(a3, p2, diswl, W2, b2[None, :])
    return out


# speedup vs baseline: 1.2512x; 1.0028x over previous
import functools

import jax
import jax.numpy as jnp
from jax import lax
from jax.experimental import pallas as pl
from jax.experimental.pallas import tpu as pltpu
from jax.experimental.pallas import tpu_sc as plsc

N = 10000
E = 320000
D = 128
NCLS = 40

_TILES = 32
_NS = 16
_EPT = E // _TILES
_CH = 80
_NCHUNK = _EPT // _CH

_ACH = 64
_W = 8
_PAD = 240
_EPTP = _EPT + _PAD
_ANCH = _EPTP // _ACH
_NG = _ANCH // _W
_NB = 5
_NP = N + _NS

_HI = jax.lax.Precision.HIGHEST


@functools.cache
def _sc_agg():
    mesh = plsc.VectorSubcoreMesh(core_axis_name="c", subcore_axis_name="s")

    @functools.partial(
        pl.kernel,
        out_type=jax.ShapeDtypeStruct((2, _NP, D), jnp.float32),
        mesh=mesh,
        scratch_types=[
            pltpu.VMEM((_W, _ACH), jnp.int32),
            pltpu.VMEM((_W, _ACH), jnp.int32),
            pltpu.VMEM((_W, _ACH), jnp.int32),
            pltpu.VMEM((_W, _ACH), jnp.int32),
            pltpu.VMEM((_ACH, D), jnp.float32),
            pltpu.VMEM((_ACH, D), jnp.float32),
            pltpu.VMEM((_ACH, D), jnp.float32),
            pltpu.VMEM((_ACH, D), jnp.float32),
            pltpu.VMEM((_ACH, D), jnp.float32),
            pltpu.VMEM_SHARED((_NP, D), jnp.float32),
            pltpu.SemaphoreType.DMA((_NB,)),
            pltpu.SemaphoreType.DMA((_NB,)),
            pltpu.SemaphoreType.DMA((2,)),
        ],
    )
    def agg(p_hbm, srcr_hbm, dstr_hbm, zmat_hbm, out_hbm,
            si0, si1, di0, di1, rb0, rb1, rb2, rb3, rb4, acc, gsem, ssem, isem):
        sibs = [si0, si1]
        dibs = [di0, di1]
        rbs = [rb0, rb1, rb2, rb3, rb4]
        c = lax.axis_index("c")
        s = lax.axis_index("s")
        t = c * _NS + s

        for i in range(_NS):
            @pl.when(s == i)
            def _():
                lo = 624 * i
                sz = _NP - lo if i == _NS - 1 else 624
                pltpu.sync_copy(zmat_hbm.at[pl.ds(lo, sz)],
                                acc.at[pl.ds(lo, sz)])

        plsc.subcore_barrier()

        def g_start(k):
            g, j, b = k // _W, k % _W, k % _NB
            pltpu.async_copy(p_hbm.at[sibs[g % 2].at[j]], rbs[b], gsem.at[b])

        def g_wait(k):
            g, j, b = k // _W, k % _W, k % _NB
            pltpu.make_async_copy(p_hbm.at[sibs[g % 2].at[j]], rbs[b],
                                  gsem.at[b]).wait()

        def s_start(k):
            g, j, b = k // _W, k % _W, k % _NB
            pltpu.async_copy(rbs[b], acc.at[dibs[g % 2].at[j]], ssem.at[b],
                             add=True)

        def s_wait(k):
            g, j, b = k // _W, k % _W, k % _NB
            pltpu.make_async_copy(rbs[b], acc.at[dibs[g % 2].at[j]],
                                  ssem.at[b]).wait()

        def i_start(g):
            ib = g % 2
            pltpu.async_copy(srcr_hbm.at[t, g], sibs[ib], isem.at[ib])
            pltpu.async_copy(dstr_hbm.at[t, g], dibs[ib], isem.at[ib])

        def i_wait(g):
            ib = g % 2
            pltpu.make_async_copy(srcr_hbm.at[t, g], sibs[ib],
                                  isem.at[ib]).wait()
            pltpu.make_async_copy(dstr_hbm.at[t, g], dibs[ib],
                                  isem.at[ib]).wait()

        pltpu.sync_copy(srcr_hbm.at[t, 0], si0)
        pltpu.sync_copy(dstr_hbm.at[t, 0], di0)
        i_start(1)
        for k in range(_NB - 1):
            g_start(k)

        for k in range(_ANCH):
            g_wait(k)
            s_start(k)
            if k > 0:
                s_wait(k - 1)
            if k % _W == 0 and k > 0 and k // _W + 1 < _NG:
                i_start(k // _W + 1)
            kn = k + _NB - 1
            if kn < _ANCH:
                if kn % _W < _NB - 1 and kn // _W > 0:
                    if kn % _W == 0:
                        i_wait(kn // _W)
                g_start(kn)

        s_wait(_ANCH - 1)
        plsc.subcore_barrier()

        for i in range(_NS):
            @pl.when(s == i)
            def _():
                lo = 624 * i
                sz = _NP - lo if i == _NS - 1 else 624
                pltpu.sync_copy(acc.at[pl.ds(lo, sz)],
                                out_hbm.at[c, pl.ds(lo, sz)])

    return agg


@functools.cache
def _sc_deg():
    mesh = plsc.VectorSubcoreMesh(core_axis_name="c", subcore_axis_name="s")

    @functools.partial(
        pl.kernel,
        out_type=jax.ShapeDtypeStruct((2, _NP), jnp.float32),
        mesh=mesh,
        scratch_types=[
            pltpu.VMEM((_W, _ACH), jnp.int32),
            pltpu.VMEM((_W, _ACH), jnp.int32),
            pltpu.VMEM((_ACH,), jnp.float32),
            pltpu.VMEM_SHARED((_NP,), jnp.float32),
            pltpu.SemaphoreType.DMA((2,)),
        ],
    )
    def deg(dstr_hbm, zvec_hbm, out_hbm, di0, di1, ones_v, acc, isem):
        dibs = [di0, di1]
        c = lax.axis_index("c")
        s = lax.axis_index("s")
        t = c * _NS + s

        @pl.when(s == 0)
        def _():
            pltpu.sync_copy(zvec_hbm, acc)

        for i in range(_ACH // 16):
            ones_v[pl.ds(i * 16, 16)] = jnp.full((16,), 1.0, jnp.float32)

        plsc.subcore_barrier()
        pltpu.sync_copy(dstr_hbm.at[t, 0], di0)

        def i_start(g):
            pltpu.async_copy(dstr_hbm.at[t, g], dibs[g % 2], isem.at[g % 2])

        def i_wait(g):
            pltpu.make_async_copy(dstr_hbm.at[t, g], dibs[g % 2],
                                  isem.at[g % 2]).wait()

        i_start(1)
        for g in range(_NG):
            if g > 0:
                i_wait(g)
            for j in range(_W):
                pltpu.sync_copy(ones_v, acc.at[dibs[g % 2].at[j]], add=True)
            if g + 2 < _NG:
                i_start(g + 2)

        plsc.subcore_barrier()

        @pl.when(s == 0)
        def _():
            pltpu.sync_copy(acc, out_hbm.at[c])

    return deg


def _tc1a_body(x_ref, w_ref, h_ref):
    h_ref[...] = jnp.dot(
        x_ref[...], w_ref[...], preferred_element_type=jnp.float32, precision=_HI
    )


def _tc1b_body(h_ref, degp_ref, p_ref, disnl_ref, diswl_ref):
    deg = degp_ref[0, :N, :] + degp_ref[1, :N, :]
    disnl = jnp.where(deg > 0, lax.rsqrt(jnp.maximum(deg, 1e-12)), 0.0)
    diswl = lax.rsqrt(deg + 1.0)
    disnl_ref[...] = disnl
    diswl_ref[...] = diswl
    p_ref[...] = disnl * h_ref[...]


def _tc2_body(a_ref, disnl_ref, b_ref, w_ref, p_ref):
    disnl = disnl_ref[...]
    x1 = jnp.maximum(
        disnl * (a_ref[0, :N, :] + a_ref[1, :N, :]) + b_ref[...], 0.0
    )
    p_ref[...] = disnl * jnp.dot(
        x1, w_ref[...], preferred_element_type=jnp.float32, precision=_HI
    )


def _tc3_body(a_ref, disnl_ref, diswl_ref, b_ref, p_ref):
    x2 = jnp.maximum(
        disnl_ref[...] * (a_ref[0, :N, :] + a_ref[1, :N, :]) + b_ref[...], 0.0
    )
    p_ref[...] = diswl_ref[...] * x2


def _tc4_body(a_ref, p2_ref, diswl_ref, w_ref, b_ref, o_ref):
    q = diswl_ref[...] * (a_ref[0, :N, :] + a_ref[1, :N, :] + p2_ref[...])
    o_ref[...] = (
        jnp.dot(q, w_ref[...], preferred_element_type=jnp.float32, precision=_HI)
        + b_ref[...]
    )


def _call(body, n_out, out_shapes):
    return pl.pallas_call(
        body,
        out_shape=[jax.ShapeDtypeStruct(s, jnp.float32) for s in out_shapes]
        if n_out > 1
        else jax.ShapeDtypeStruct(out_shapes[0], jnp.float32),
    )


def kernel(x, edge_index, W0, b0, W1, b1, W2, b2):
    tvec = jnp.arange(_TILES, dtype=jnp.int32)[:, None]
    evec = jnp.arange(_PAD, dtype=jnp.int32)[None, :]
    dsrc = (tvec * 313 + evec * 41) % N
    ddst = N + (tvec + evec) % _NS
    srcr = jnp.concatenate(
        [edge_index[0].reshape(_TILES, _EPT), dsrc], axis=1
    ).reshape(_TILES, _NG, _W, _ACH)
    dstr = jnp.concatenate(
        [edge_index[1].reshape(_TILES, _EPT), ddst], axis=1
    ).reshape(_TILES, _NG, _W, _ACH)
    zvec = jnp.zeros((_NP,), jnp.float32)
    zmat = jnp.zeros((_NP, D), jnp.float32)

    degp = _sc_deg()(dstr, zvec)
    h0 = _call(_tc1a_body, 1, [(N, D)])(x, W0)

    p0, disnl, diswl = _call(_tc1b_body, 3, [(N, D), (N, 1), (N, 1)])(
        h0, degp.reshape(2, _NP, 1)
    )

    a1 = _sc_agg()(p0, srcr, dstr, zmat)
    p1 = _call(_tc2_body, 1, [(N, D)])(a1, disnl, b0[None, :], W1)

    a2 = _sc_agg()(p1, srcr, dstr, zmat)
    p2 = _call(_tc3_body, 1, [(N, D)])(a2, disnl, diswl, b1[None, :])

    a3 = _sc_agg()(p2, srcr, dstr, zmat)
    out = _call(_tc4_body, 1, [(N, NCLS)])(a3, p2, diswl, W2, b2[None, :])
    return out
